# baseline (speedup 1.0000x reference)
import jax
import jax.numpy as jnp
from jax import lax
from jax.experimental import pallas as pl
from jax.experimental.pallas import tpu as pltpu
from jax.experimental.pallas import tpu_sc as plsc

N = 10000
E = 320000
D = 128
NC = 2
NS = 16
NW = NC * NS
CH = 125
NCHUNK = E // CH
CPW = NCHUNK // NW
IPH = 40
NP = 10240
RPT = NP // NS
DEGW = 16


def _agg_body(x_hbm, src_hbm, dst_hbm, zrow_hbm,
              agg_out,
              src_v, dst_v, rows_v, acc_sh, sem0):
    c = lax.axis_index("c")
    s = lax.axis_index("s")
    wid = s * NC + c

    pltpu.sync_copy(zrow_hbm.at[pl.ds(s * RPT, RPT)],
                    acc_sh.at[pl.ds(s * RPT, RPT)])
    plsc.subcore_barrier()

    def phase(p, _):
        base = wid * CPW + p * IPH
        pltpu.sync_copy(src_hbm.at[pl.ds(base, IPH)], src_v)
        pltpu.sync_copy(dst_hbm.at[pl.ds(base, IPH)], dst_v)

        @pl.when((s % 2 == 1) & (p == 0))
        def _():
            pltpu.async_copy(x_hbm.at[src_v.at[0]], rows_v, sem0).wait()

        def step(j, _):
            pltpu.async_copy(x_hbm.at[src_v.at[j]], rows_v, sem0).wait()
            pltpu.sync_copy(rows_v, acc_sh.at[dst_v.at[j]], add=True)
            return 0
        lax.fori_loop(0, IPH, step, 0)
        return 0
    lax.fori_loop(0, CPW // IPH, phase, 0)

    plsc.subcore_barrier()

    pltpu.sync_copy(acc_sh.at[pl.ds(s * RPT, RPT)],
                    agg_out.at[c, pl.ds(s * RPT, RPT)])


_sc_aggregate = pl.kernel(
    _agg_body,
    out_type=jax.ShapeDtypeStruct((NC, NP, D), jnp.float32),
    mesh=plsc.VectorSubcoreMesh(core_axis_name="c", subcore_axis_name="s"),
    scratch_types=[
        pltpu.VMEM((IPH, CH), jnp.int32),
        pltpu.VMEM((IPH, CH), jnp.int32),
        pltpu.VMEM((CH, D), jnp.float32),
        pltpu.VMEM_SHARED((NP, D), jnp.float32),
        pltpu.SemaphoreType.DMA,
    ],
)


def _deg_body(dst_hbm, zdeg_hbm, deg_out, dst_v, ones_v, deg_sh):
    c = lax.axis_index("c")
    s = lax.axis_index("s")
    wid = s * NC + c

    pltpu.sync_copy(dst_hbm.at[pl.ds(wid * CPW, CPW)], dst_v)

    def init_ones(i, _):
        ones_v[i] = jnp.ones((DEGW,), jnp.float32)
        return 0
    lax.fori_loop(0, CH, init_ones, 0)

    pltpu.sync_copy(zdeg_hbm.at[pl.ds(s * RPT, RPT)],
                    deg_sh.at[pl.ds(s * RPT, RPT)])
    plsc.subcore_barrier()

    def step(j, _):
        pltpu.sync_copy(ones_v, deg_sh.at[dst_v.at[j]], add=True)
        return 0
    lax.fori_loop(0, CPW, step, 0)

    plsc.subcore_barrier()
    pltpu.sync_copy(deg_sh.at[pl.ds(s * RPT, RPT)],
                    deg_out.at[c, pl.ds(s * RPT, RPT)])


_sc_degree = pl.kernel(
    _deg_body,
    out_type=jax.ShapeDtypeStruct((NC, NP, DEGW), jnp.float32),
    mesh=plsc.VectorSubcoreMesh(core_axis_name="c", subcore_axis_name="s"),
    scratch_types=[
        pltpu.VMEM((CPW, CH), jnp.int32),
        pltpu.VMEM((CH, DEGW), jnp.float32),
        pltpu.VMEM_SHARED((NP, DEGW), jnp.float32),
    ],
)


ROWS_BLK = 1000


def _lin_body(agg_ref, deg_ref, x_ref, wn_ref, ws_ref, b_ref, o_ref):
    a = agg_ref[0] + agg_ref[1]
    deg = deg_ref[0, :, :1] + deg_ref[1, :, :1]
    mean = a / jnp.maximum(deg, 1.0)
    o_ref[...] = (
        jnp.dot(mean, wn_ref[...], preferred_element_type=jnp.float32)
        + jnp.dot(x_ref[...], ws_ref[...], preferred_element_type=jnp.float32)
        + b_ref[...]
    )


def _tc_linear(agg, deg, x, w_neigh, w_self, b):
    grid = N // ROWS_BLK
    return pl.pallas_call(
        _lin_body,
        grid=(grid,),
        in_specs=[
            pl.BlockSpec((NC, ROWS_BLK, D), lambda i: (0, i, 0)),
            pl.BlockSpec((NC, ROWS_BLK, DEGW), lambda i: (0, i, 0)),
            pl.BlockSpec((ROWS_BLK, D), lambda i: (i, 0)),
            pl.BlockSpec((D, D), lambda i: (0, 0)),
            pl.BlockSpec((D, D), lambda i: (0, 0)),
            pl.BlockSpec((D,), lambda i: (0,)),
        ],
        out_specs=pl.BlockSpec((ROWS_BLK, D), lambda i: (i, 0)),
        out_shape=jax.ShapeDtypeStruct((N, D), jnp.float32),
    )(agg, deg, x, w_neigh, w_self, b)


def kernel(x, edge_index, W1_neigh, W1_self, b1, W2_neigh, W2_self, b2):
    src2 = edge_index[0].astype(jnp.int32).reshape(NCHUNK, CH)
    dst2 = edge_index[1].astype(jnp.int32).reshape(NCHUNK, CH)
    zrow = jnp.zeros((NP, D), jnp.float32)
    zdeg = jnp.zeros((NP, DEGW), jnp.float32)

    deg = _sc_degree(dst2, zdeg)
    agg1 = _sc_aggregate(x, src2, dst2, zrow)
    h = _tc_linear(agg1, deg, x, W1_neigh, W1_self, b1)
    agg2 = _sc_aggregate(h, src2, dst2, zrow)
    out = _tc_linear(agg2, deg, h, W2_neigh, W2_self, b2)
    return out


# speedup vs baseline: 1.4227x; 1.4227x over previous
import jax
import jax.numpy as jnp
from jax import lax
from jax.experimental import pallas as pl
from jax.experimental.pallas import tpu as pltpu
from jax.experimental.pallas import tpu_sc as plsc

N = 10000
E = 320000
D = 128
NC = 2
NS = 16
NW = NC * NS
CH = 125
NCHUNK = E // CH
CPW = NCHUNK // NW
IPH = 40
NP = 10240
RPT = NP // NS
DEGW = 16


def _agg_body(x_hbm, src_hbm, dst_hbm, zrow_hbm,
              agg_out,
              src_v, dst_v, rows_v0, rows_v1, acc_sh, sem0, sem1):
    c = lax.axis_index("c")
    s = lax.axis_index("s")
    wid = s * NC + c

    pltpu.sync_copy(zrow_hbm.at[pl.ds(s * RPT, RPT)],
                    acc_sh.at[pl.ds(s * RPT, RPT)])
    plsc.subcore_barrier()

    def phase(p, _):
        base = wid * CPW + p * IPH
        pltpu.sync_copy(src_hbm.at[pl.ds(base, IPH)], src_v)
        pltpu.sync_copy(dst_hbm.at[pl.ds(base, IPH)], dst_v)
        pltpu.async_copy(x_hbm.at[src_v.at[0]], rows_v0, sem0)

        def step(j2, _):
            j = j2 * 2
            pltpu.async_copy(x_hbm.at[src_v.at[j + 1]], rows_v1, sem1)
            pltpu.make_async_copy(x_hbm.at[src_v.at[j]], rows_v0, sem0).wait()
            pltpu.sync_copy(rows_v0, acc_sh.at[dst_v.at[j]], add=True)
            nxt = jnp.minimum(j + 2, IPH - 1)
            pltpu.async_copy(x_hbm.at[src_v.at[nxt]], rows_v0, sem0)
            pltpu.make_async_copy(
                x_hbm.at[src_v.at[j + 1]], rows_v1, sem1).wait()
            pltpu.sync_copy(rows_v1, acc_sh.at[dst_v.at[j + 1]], add=True)
            return 0
        lax.fori_loop(0, IPH // 2, step, 0)
        pltpu.make_async_copy(x_hbm.at[src_v.at[0]], rows_v0, sem0).wait()
        return 0
    lax.fori_loop(0, CPW // IPH, phase, 0)

    plsc.subcore_barrier()

    pltpu.sync_copy(acc_sh.at[pl.ds(s * RPT, RPT)],
                    agg_out.at[c, pl.ds(s * RPT, RPT)])


_sc_aggregate = pl.kernel(
    _agg_body,
    out_type=jax.ShapeDtypeStruct((NC, NP, D), jnp.float32),
    mesh=plsc.VectorSubcoreMesh(core_axis_name="c", subcore_axis_name="s"),
    scratch_types=[
        pltpu.VMEM((IPH, CH), jnp.int32),
        pltpu.VMEM((IPH, CH), jnp.int32),
        pltpu.VMEM((CH, D), jnp.float32),
        pltpu.VMEM((CH, D), jnp.float32),
        pltpu.VMEM_SHARED((NP, D), jnp.float32),
        pltpu.SemaphoreType.DMA,
        pltpu.SemaphoreType.DMA,
    ],
)


def _deg_body(dst_hbm, zdeg_hbm, deg_out, dst_v, ones_v, deg_sh):
    c = lax.axis_index("c")
    s = lax.axis_index("s")
    wid = s * NC + c

    pltpu.sync_copy(dst_hbm.at[pl.ds(wid * CPW, CPW)], dst_v)

    def init_ones(i, _):
        ones_v[i] = jnp.ones((DEGW,), jnp.float32)
        return 0
    lax.fori_loop(0, CH, init_ones, 0)

    pltpu.sync_copy(zdeg_hbm.at[pl.ds(s * RPT, RPT)],
                    deg_sh.at[pl.ds(s * RPT, RPT)])
    plsc.subcore_barrier()

    def step(j, _):
        pltpu.sync_copy(ones_v, deg_sh.at[dst_v.at[j]], add=True)
        return 0
    lax.fori_loop(0, CPW, step, 0)

    plsc.subcore_barrier()
    pltpu.sync_copy(deg_sh.at[pl.ds(s * RPT, RPT)],
                    deg_out.at[c, pl.ds(s * RPT, RPT)])


_sc_degree = pl.kernel(
    _deg_body,
    out_type=jax.ShapeDtypeStruct((NC, NP, DEGW), jnp.float32),
    mesh=plsc.VectorSubcoreMesh(core_axis_name="c", subcore_axis_name="s"),
    scratch_types=[
        pltpu.VMEM((CPW, CH), jnp.int32),
        pltpu.VMEM((CH, DEGW), jnp.float32),
        pltpu.VMEM_SHARED((NP, DEGW), jnp.float32),
    ],
)


ROWS_BLK = 1000


def _lin_body(agg_ref, deg_ref, x_ref, wn_ref, ws_ref, b_ref, o_ref):
    a = agg_ref[0] + agg_ref[1]
    deg = deg_ref[0, :, :1] + deg_ref[1, :, :1]
    mean = a / jnp.maximum(deg, 1.0)
    o_ref[...] = (
        jnp.dot(mean, wn_ref[...], preferred_element_type=jnp.float32)
        + jnp.dot(x_ref[...], ws_ref[...], preferred_element_type=jnp.float32)
        + b_ref[...]
    )


def _tc_linear(agg, deg, x, w_neigh, w_self, b):
    grid = N // ROWS_BLK
    return pl.pallas_call(
        _lin_body,
        grid=(grid,),
        in_specs=[
            pl.BlockSpec((NC, ROWS_BLK, D), lambda i: (0, i, 0)),
            pl.BlockSpec((NC, ROWS_BLK, DEGW), lambda i: (0, i, 0)),
            pl.BlockSpec((ROWS_BLK, D), lambda i: (i, 0)),
            pl.BlockSpec((D, D), lambda i: (0, 0)),
            pl.BlockSpec((D, D), lambda i: (0, 0)),
            pl.BlockSpec((D,), lambda i: (0,)),
        ],
        out_specs=pl.BlockSpec((ROWS_BLK, D), lambda i: (i, 0)),
        out_shape=jax.ShapeDtypeStruct((N, D), jnp.float32),
    )(agg, deg, x, w_neigh, w_self, b)


def kernel(x, edge_index, W1_neigh, W1_self, b1, W2_neigh, W2_self, b2):
    src2 = edge_index[0].astype(jnp.int32).reshape(NCHUNK, CH)
    dst2 = edge_index[1].astype(jnp.int32).reshape(NCHUNK, CH)
    zrow = jnp.zeros((NP, D), jnp.float32)
    zdeg = jnp.zeros((NP, DEGW), jnp.float32)

    deg = _sc_degree(dst2, zdeg)
    agg1 = _sc_aggregate(x, src2, dst2, zrow)
    h = _tc_linear(agg1, deg, x, W1_neigh, W1_self, b1)
    agg2 = _sc_aggregate(h, src2, dst2, zrow)
    out = _tc_linear(agg2, deg, h, W2_neigh, W2_self, b2)
    return out
